# baseline (speedup 1.0000x reference)
import functools
import math

import jax
import jax.numpy as jnp
from jax import lax
from jax.experimental import pallas as pl
from jax.experimental.pallas import tpu as pltpu
from jax.experimental.pallas import tpu_sc as plsc

_B = 4096 * 50
_D = 32
_NC, _NS = 2, 16
_NW = _NC * _NS
_BPW = _B // _NW
_CHUNK = 3200
_NCHUNK = _BPW // _CHUNK
_SQRT_S = math.sqrt(32.0)
_LANES = 16


@functools.partial(
    pl.kernel,
    mesh=plsc.VectorSubcoreMesh(core_axis_name="c", subcore_axis_name="s"),
    compiler_params=pltpu.CompilerParams(use_tc_tiling_on_sc=False),
    out_type=jax.ShapeDtypeStruct((_B, _D), jnp.float32),
    scratch_types=[
        pltpu.VMEM((_BPW,), jnp.int32),
        pltpu.VMEM((_CHUNK, _D), jnp.float32),
        pltpu.SemaphoreType.DMA,
    ],
)
def _embed_sc(x_hbm, table_hbm, out_hbm, idx_v, rows_v, sem):
    wid = lax.axis_index("s") * _NC + lax.axis_index("c")
    base = wid * _BPW
    pltpu.sync_copy(x_hbm.at[pl.ds(base, _BPW)], idx_v)
    for c in range(_NCHUNK):
        off = c * _CHUNK
        pltpu.async_copy(
            table_hbm.at[idx_v.at[pl.ds(off, _CHUNK)]], rows_v, sem
        ).wait()

        def scale_row(i, carry):
            rows_v[i, pl.ds(0, _LANES)] = rows_v[i, pl.ds(0, _LANES)] * _SQRT_S
            rows_v[i, pl.ds(_LANES, _LANES)] = (
                rows_v[i, pl.ds(_LANES, _LANES)] * _SQRT_S
            )
            return carry

        lax.fori_loop(0, _CHUNK, scale_row, 0)
        pltpu.sync_copy(rows_v, out_hbm.at[pl.ds(base + off, _CHUNK)])


def kernel(x, table):
    b, l = x.shape
    flat_idx = x.reshape(-1).astype(jnp.int32)
    out = _embed_sc(flat_idx, table)
    return out.reshape(b, l, _D)


# speedup vs baseline: 1.1518x; 1.1518x over previous
import functools
import math

import jax
import jax.numpy as jnp
from jax import lax
from jax.experimental import pallas as pl
from jax.experimental.pallas import tpu as pltpu
from jax.experimental.pallas import tpu_sc as plsc

_BATCH = 4096
_LSEQ = 50
_D = 32
_NC, _NS = 2, 16
_NW = _NC * _NS
_BPW = _BATCH // _NW
_LOOK = _BPW * _LSEQ
_LCH = 10
_NCH = _LSEQ // _LCH
_CLOOK = _BPW * _LCH
_SQRT_S = math.sqrt(32.0)
_L = 16


@functools.partial(
    pl.kernel,
    mesh=plsc.VectorSubcoreMesh(core_axis_name="c", subcore_axis_name="s"),
    compiler_params=pltpu.CompilerParams(
        use_tc_tiling_on_sc=False, needs_layout_passes=False
    ),
    out_type=jax.ShapeDtypeStruct((_LSEQ, 4, _NW, 8, _BPW), jnp.float32),
    scratch_types=[
        pltpu.VMEM((_BPW, _LSEQ), jnp.int32),
        pltpu.VMEM((_LOOK,), jnp.int32),
        pltpu.VMEM((_CLOOK, _D), jnp.float32),
        pltpu.VMEM((_LCH, 4, 1, 8, _BPW), jnp.float32),
        pltpu.SemaphoreType.DMA,
    ],
)
def _embed_sc(x_hbm, table_hbm, out_hbm, idx_v, idx1_v, rows_v, outc_v, sem):
    wid = lax.axis_index("s") * _NC + lax.axis_index("c")
    b0 = wid * _BPW
    pltpu.sync_copy(x_hbm.at[pl.ds(b0, _BPW), :], idx_v)

    lanes = lax.iota(jnp.int32, _L)

    def tr_body(g, carry):
        li = lax.shift_right_logical(g, 3)
        cg = jnp.bitwise_and(g, 7)
        rows16 = lanes + cg * _L
        vals = plsc.load_gather(idx_v, [rows16, jnp.broadcast_to(li, (_L,))])
        idx1_v[pl.ds(li * _BPW + cg * _L, _L)] = vals
        return carry

    lax.fori_loop(0, (_LOOK // _L), tr_body, 0)

    for c in range(_NCH):
        l0 = c * _LCH
        pltpu.async_copy(
            table_hbm.at[idx1_v.at[pl.ds(l0 * _BPW, _CLOOK)]], rows_v, sem
        ).wait()

        for d in range(_D):
            td, r = d >> 3, d & 7

            def re_body(g, carry, td=td, r=r, d=d):
                li = lax.shift_right_logical(g, 3)
                cg = jnp.bitwise_and(g, 7)
                rows16 = li * _BPW + cg * _L + lanes
                vals = plsc.load_gather(
                    rows_v, [rows16, jnp.broadcast_to(jnp.int32(d), (_L,))]
                )
                outc_v[li, td, 0, r, pl.ds(cg * _L, _L)] = vals * _SQRT_S
                return carry

            lax.fori_loop(0, (_CLOOK // _L), re_body, 0)

        pltpu.sync_copy(
            outc_v,
            out_hbm.at[pl.ds(l0, _LCH), pl.ds(0, 4), pl.ds(wid, 1)],
        )


def kernel(x, table):
    x2 = x.astype(jnp.int32)
    out5 = _embed_sc(x2, table)
    out = out5.transpose(2, 4, 0, 1, 3).reshape(_BATCH, _LSEQ, _D)
    return out


# speedup vs baseline: 1.3566x; 1.1778x over previous
import functools
import math

import jax
import jax.numpy as jnp
from jax import lax
from jax.experimental import pallas as pl
from jax.experimental.pallas import tpu as pltpu
from jax.experimental.pallas import tpu_sc as plsc

_BATCH = 4096
_LSEQ = 50
_D = 32
_NC, _NS = 2, 16
_NW = _NC * _NS
_BPW = _BATCH // _NW
_LOOK = _BPW * _LSEQ
_LCH = 5
_NCH = _LSEQ // _LCH
_CLOOK = _BPW * _LCH
_SQRT_S = math.sqrt(32.0)
_L = 16
_GROUPS = _CLOOK // _L


@functools.partial(
    pl.kernel,
    mesh=plsc.VectorSubcoreMesh(core_axis_name="c", subcore_axis_name="s"),
    compiler_params=pltpu.CompilerParams(
        use_tc_tiling_on_sc=False, needs_layout_passes=False
    ),
    out_type=jax.ShapeDtypeStruct((_LSEQ, 4, _NW, 8, _BPW), jnp.float32),
    scratch_types=[
        pltpu.VMEM((_BPW, _LSEQ), jnp.int32),
        pltpu.VMEM((_LOOK,), jnp.int32),
        pltpu.VMEM((2, _CLOOK, _D), jnp.float32),
        pltpu.VMEM((2, _LCH, 4, 1, 8, _BPW), jnp.float32),
        pltpu.SemaphoreType.DMA,
        pltpu.SemaphoreType.DMA,
        pltpu.SemaphoreType.DMA,
        pltpu.SemaphoreType.DMA,
    ],
)
def _embed_sc(
    x_hbm, table_hbm, out_hbm, idx_v, idx1_v, rows_v, outc_v, sg0, sg1, ss0, ss1
):
    wid = lax.axis_index("s") * _NC + lax.axis_index("c")
    b0 = wid * _BPW
    pltpu.sync_copy(x_hbm.at[pl.ds(b0, _BPW), :], idx_v)

    lanes = lax.iota(jnp.int32, _L)

    @plsc.parallel_loop(0, _LOOK // _L, unroll=4)
    def _(g):
        li = lax.shift_right_logical(g, 3)
        cg = jnp.bitwise_and(g, 7)
        rows16 = lanes + cg * _L
        vals = plsc.load_gather(idx_v, [rows16, jnp.broadcast_to(li, (_L,))])
        idx1_v[pl.ds(li * _BPW + cg * _L, _L)] = vals

    gsems = (sg0, sg1)
    ssems = (ss0, ss1)

    def start_gather(c):
        return pltpu.async_copy(
            table_hbm.at[idx1_v.at[pl.ds(c * _CLOOK, _CLOOK)]],
            rows_v.at[c % 2],
            gsems[c % 2],
        )

    gather_descs = [None] * _NCH
    store_descs = [None] * _NCH
    gather_descs[0] = start_gather(0)
    for c in range(_NCH):
        if c + 1 < _NCH:
            gather_descs[c + 1] = start_gather(c + 1)
        gather_descs[c].wait()
        if c >= 2:
            store_descs[c - 2].wait()
        rbuf = rows_v.at[c % 2]
        obuf = outc_v.at[c % 2]

        @plsc.parallel_loop(0, _GROUPS, unroll=1)
        def _(g):
            li = lax.shift_right_logical(g, 3)
            cg = jnp.bitwise_and(g, 7)
            rows16 = li * _BPW + cg * _L + lanes
            cslice = pl.ds(cg * _L, _L)
            for d in range(_D):
                vals = plsc.load_gather(
                    rbuf, [rows16, jnp.broadcast_to(jnp.int32(d), (_L,))]
                )
                obuf[li, d >> 3, 0, d & 7, cslice] = vals * _SQRT_S

        store_descs[c] = pltpu.async_copy(
            obuf,
            out_hbm.at[pl.ds(c * _LCH, _LCH), pl.ds(0, 4), pl.ds(wid, 1)],
            ssems[c % 2],
        )
    store_descs[_NCH - 2].wait()
    store_descs[_NCH - 1].wait()


def kernel(x, table):
    x2 = x.astype(jnp.int32)
    out5 = _embed_sc(x2, table)
    out = out5.transpose(2, 4, 0, 1, 3).reshape(_BATCH, _LSEQ, _D)
    return out
